# baseline (speedup 1.0000x reference)
import functools

import jax
import jax.numpy as jnp
from jax import lax
from jax.experimental import pallas as pl
from jax.experimental.pallas import tpu as pltpu
from jax.experimental.pallas import tpu_sc as plsc

N = 10000
E = 320000
D = 128
R = 4
B = 512
NC = 2
NS = 16
L = 16
NW = NC * NS
EPT = E // NW
C = 80
NCHUNK = EPT // C
ROW_STRIDE = 624
ROW_SPAN = 640
CNT_SLICE = (4 * N) // 10
EPC = E // NS
NCHUNK_CNT = EPC // C

_mesh = plsc.VectorSubcoreMesh(core_axis_name="c", subcore_axis_name="s")


@functools.partial(
    pl.kernel,
    mesh=_mesh,
    out_type=jax.ShapeDtypeStruct((4 * N,), jnp.float32),
    scratch_types=[
        pltpu.VMEM((NCHUNK_CNT, C), jnp.int32),
        pltpu.VMEM((C,), jnp.float32),
        pltpu.VMEM((CNT_SLICE,), jnp.float32),
        pltpu.VMEM_SHARED((4 * N,), jnp.float32),
    ],
)
def _count_kernel(key_hbm, inv_hbm, keys_v, ones_v, buf_v, cnt_sh):
    s = lax.axis_index("s")

    def zero_body(i, _):
        buf_v[pl.ds(i * L, L)] = jnp.zeros((L,), jnp.float32)
        return 0

    lax.fori_loop(0, CNT_SLICE // L, zero_body, 0)

    @pl.when(s < 10)
    def _():
        pltpu.sync_copy(buf_v, cnt_sh.at[pl.ds(s * CNT_SLICE, CNT_SLICE)])

    for i in range(C // L):
        ones_v[pl.ds(i * L, L)] = jnp.ones((L,), jnp.float32)
    pltpu.sync_copy(key_hbm.at[s], keys_v)
    plsc.subcore_barrier()

    def chunk(k, _):
        pltpu.sync_copy(ones_v, cnt_sh.at[keys_v.at[k]], add=True)
        return 0

    lax.fori_loop(0, NCHUNK_CNT, chunk, 0)
    plsc.subcore_barrier()

    @pl.when(s < 10)
    def _():
        pltpu.sync_copy(cnt_sh.at[pl.ds(s * CNT_SLICE, CNT_SLICE)], buf_v)

        def inv_body(i, _):
            sl = pl.ds(i * L, L)
            buf_v[sl] = 1.0 / jnp.maximum(buf_v[sl], 1.0)
            return 0

        lax.fori_loop(0, CNT_SLICE // L, inv_body, 0)
        pltpu.sync_copy(buf_v, inv_hbm.at[pl.ds(s * CNT_SLICE, CNT_SLICE)])


@functools.partial(
    pl.kernel,
    mesh=_mesh,
    out_type=jax.ShapeDtypeStruct((NC, N, D), jnp.float32),
    scratch_types=[
        pltpu.VMEM((NCHUNK, C), jnp.int32),
        pltpu.VMEM((C,), jnp.int32), pltpu.VMEM((C,), jnp.int32),
        pltpu.VMEM((C,), jnp.int32),
        pltpu.VMEM((C,), jnp.int32), pltpu.VMEM((C,), jnp.int32),
        pltpu.VMEM((C,), jnp.int32),
        pltpu.VMEM((C,), jnp.int32), pltpu.VMEM((C,), jnp.int32),
        pltpu.VMEM((C,), jnp.int32),
        pltpu.VMEM((C,), jnp.float32), pltpu.VMEM((C,), jnp.float32),
        pltpu.VMEM((C,), jnp.float32),
        pltpu.VMEM((C, D), jnp.float32), pltpu.VMEM((C, D), jnp.float32),
        pltpu.VMEM((C, D), jnp.float32),
        pltpu.VMEM_SHARED((N, D), jnp.float32),
        pltpu.SemaphoreType.DMA, pltpu.SemaphoreType.DMA,
        pltpu.SemaphoreType.DMA,
        pltpu.SemaphoreType.DMA, pltpu.SemaphoreType.DMA,
        pltpu.SemaphoreType.DMA,
        pltpu.SemaphoreType.DMA, pltpu.SemaphoreType.DMA,
        pltpu.SemaphoreType.DMA,
    ],
)
def _edge_kernel(y_hbm, pidx_hbm, inv_hbm, out_hbm,
                 pidx_v, ridx0, ridx1, ridx2, dst0, dst1, dst2,
                 key0, key1, key2, svl0, svl1, svl2,
                 rows0, rows1, rows2, acc_sh,
                 g0, g1, g2, v0, v1, v2, s0, s1, s2):
    c = lax.axis_index("c")
    s = lax.axis_index("s")
    wid = c * NS + s
    RIDX = (ridx0, ridx1, ridx2)
    DST = (dst0, dst1, dst2)
    KEY = (key0, key1, key2)
    SVL = (svl0, svl1, svl2)
    ROWS = (rows0, rows1, rows2)
    G = (g0, g1, g2)
    V = (v0, v1, v2)
    SS = (s0, s1, s2)

    def zero_body(i, _):
        rows0[i >> 3, pl.ds((i & 7) * L, L)] = jnp.zeros((L,), jnp.float32)
        return 0

    lax.fori_loop(0, C * (D // L), zero_body, 0)
    for t in range(ROW_SPAN // C):
        pltpu.sync_copy(rows0,
                        acc_sh.at[pl.ds(s * ROW_STRIDE + t * C, C)])

    pltpu.sync_copy(pidx_hbm.at[wid], pidx_v)
    plsc.subcore_barrier()

    def launch(k, b):
        def group(g, _):
            sl = pl.ds(g * L, L)
            p16 = pidx_v[k, sl]
            d16 = lax.bitwise_and(p16, (1 << 14) - 1)
            r16 = lax.shift_right_logical(p16, 14)
            RIDX[b][sl] = r16
            DST[b][sl] = d16
            KEY[b][sl] = 4 * d16 + lax.bitwise_and(r16, 3)
            return 0

        lax.fori_loop(0, C // L, group, 0)
        pltpu.async_copy(y_hbm.at[RIDX[b]], ROWS[b], G[b])
        pltpu.async_copy(inv_hbm.at[KEY[b]], SVL[b], V[b])

    def wait_g(b):
        pltpu.make_async_copy(y_hbm.at[RIDX[b]], ROWS[b], G[b]).wait()

    def wait_v(b):
        pltpu.make_async_copy(inv_hbm.at[KEY[b]], SVL[b], V[b]).wait()

    def wait_s(b):
        pltpu.make_async_copy(ROWS[b], acc_sh.at[DST[b]], SS[b]).wait()

    def scale(b):
        def group(g, _):
            sv16 = SVL[b][pl.ds(g * L, L)]
            for e in range(L):
                sb = jnp.broadcast_to(sv16[e], (L,))
                row = g * L + e
                for j in range(D // L):
                    sl = pl.ds(j * L, L)
                    ROWS[b][row, sl] = ROWS[b][row, sl] * sb
            return 0

        lax.fori_loop(0, C // L, group, 0)

    def step(j, b, bn, first_steps):
        if not first_steps:
            wait_s(bn)

        @pl.when(j + 1 < NCHUNK)
        def _():
            launch(j + 1, bn)

        wait_g(b)
        wait_v(b)
        scale(b)
        pltpu.async_copy(ROWS[b], acc_sh.at[DST[b]], SS[b], add=True)

    launch(0, 0)

    def triple(m, _):
        j = 3 * m
        for jj in range(3):
            step(j + jj, jj, (jj + 1) % 3, False)
        return 0

    for jj in range(3):
        step(jj, jj, (jj + 1) % 3, jj < 2)
    lax.fori_loop(1, (NCHUNK - 2) // 3, triple, 0)

    wait_s(1)
    launch(NCHUNK - 1, 1)
    wait_g(0)
    wait_v(0)
    scale(0)
    pltpu.async_copy(ROWS[0], acc_sh.at[DST[0]], SS[0], add=True)

    wait_s(2)
    wait_g(1)
    wait_v(1)
    scale(1)
    pltpu.sync_copy(ROWS[1], acc_sh.at[DST[1]], add=True)
    wait_s(0)

    plsc.subcore_barrier()
    pltpu.sync_copy(acc_sh.at[pl.ds(s * ROW_STRIDE, ROW_SPAN)],
                    out_hbm.at[c, pl.ds(s * ROW_STRIDE, ROW_SPAN)])


POOL_CHUNKS = N // C
POOL_ITERS = -(-POOL_CHUNKS // NW)
B_PER_TILE = B // NS


@functools.partial(
    pl.kernel,
    mesh=_mesh,
    out_type=jax.ShapeDtypeStruct((NC, B, D), jnp.float32),
    scratch_types=[
        pltpu.VMEM((C,), jnp.int32),
        pltpu.VMEM((C, D), jnp.float32),
        pltpu.VMEM_SHARED((B, D), jnp.float32),
    ],
)
def _pool_kernel(h_hbm, batch_hbm, out_hbm, bat_v, rows_v, acc_sh):
    c = lax.axis_index("c")
    s = lax.axis_index("s")
    wid = c * NS + s

    def zero_body(i, _):
        rows_v[i >> 3, pl.ds((i & 7) * L, L)] = jnp.zeros((L,), jnp.float32)
        return 0

    lax.fori_loop(0, B_PER_TILE * (D // L), zero_body, 0)
    pltpu.sync_copy(rows_v.at[pl.ds(0, B_PER_TILE)],
                    acc_sh.at[pl.ds(s * B_PER_TILE, B_PER_TILE)])
    plsc.subcore_barrier()

    def body(k, _):
        cid = k * NW + wid

        @pl.when(cid < POOL_CHUNKS)
        def _():
            base = cid * C
            pltpu.sync_copy(batch_hbm.at[pl.ds(base, C)], bat_v)
            pltpu.sync_copy(h_hbm.at[pl.ds(base, C)], rows_v)
            pltpu.sync_copy(rows_v, acc_sh.at[bat_v], add=True)

        return 0

    lax.fori_loop(0, POOL_ITERS, body, 0)
    plsc.subcore_barrier()
    pltpu.sync_copy(acc_sh.at[pl.ds(s * B_PER_TILE, B_PER_TILE)],
                    out_hbm.at[c, pl.ds(s * B_PER_TILE, B_PER_TILE)])


BLK = 1000
GRID = N // BLK


def _silu(v):
    return v / (1.0 + jnp.exp(-v))


def _mm(a, w):
    return jnp.dot(a, w, preferred_element_type=jnp.float32)


def _k0_body(x_ref, wr_ref, b_ref, wa_ref, pre_ref, y_ref):
    xb = x_ref[...]
    pre_ref[...] = _mm(xb, wr_ref[...]) + b_ref[...]
    y_ref[...] = _mm(xb, wa_ref[...])


def _k1_body(pr_ref, p0_ref, p1_ref, g_ref, bb_ref, wr_ref, b1_ref, wa_ref,
             y_ref, pr2_ref, pre_buf, acc_ref):
    p = pl.program_id(0)
    i = pl.program_id(1)

    @pl.when(jnp.logical_and(p == 0, i == 0))
    def _():
        acc_ref[...] = jnp.zeros_like(acc_ref)

    @pl.when(p == 0)
    def _():
        v = pr_ref[...] + p0_ref[...] + p1_ref[...]
        pre_buf[pl.ds(i * BLK, BLK), :] = v
        s1 = jnp.sum(v, axis=0, keepdims=True)
        s2 = jnp.sum(v * v, axis=0, keepdims=True)
        acc_ref[...] = acc_ref[...] + jnp.concatenate([s1, s2], axis=0)

    @pl.when(p == 1)
    def _():
        st = acc_ref[...]
        m = st[0:1] / N
        var = st[1:2] / N - m * m
        v = ((pre_buf[pl.ds(i * BLK, BLK), :] - m) / jnp.sqrt(var + 1e-5)
             * g_ref[...] + bb_ref[...])
        h = _silu(v)
        y_ref[...] = _mm(h, wa_ref[...])
        pr2_ref[...] = _mm(h, wr_ref[...]) + b1_ref[...]


def _k2_body(pr_ref, p0_ref, p1_ref, wr_ref, b_ref, wa_ref, y_ref, pr2_ref):
    h = _silu(pr_ref[...] + p0_ref[...] + p1_ref[...])
    y_ref[...] = _mm(h, wa_ref[...])
    pr2_ref[...] = _mm(h, wr_ref[...]) + b_ref[...]


def _k3_body(pr_ref, p0_ref, p1_ref, h_ref):
    h_ref[...] = _silu(pr_ref[...] + p0_ref[...] + p1_ref[...])


def _head_body(hg0_ref, hg1_ref, mf_ref, wm0_ref, bm0_ref, gm_ref, bbm_ref,
               wm1_ref, bm1_ref, wf0_ref, bf0_ref, wf1_ref, bf1_ref,
               wf2_ref, bf2_ref, out_ref):
    hg = hg0_ref[...] + hg1_ref[...]
    hm = _mm(mf_ref[...], wm0_ref[...]) + bm0_ref[...]
    m = jnp.mean(hm, axis=0, keepdims=True)
    var = jnp.mean((hm - m) ** 2, axis=0, keepdims=True)
    hm = (hm - m) / jnp.sqrt(var + 1e-5) * gm_ref[...] + bbm_ref[...]
    hm = _silu(hm)
    hm = _silu(_mm(hm, wm1_ref[...]) + bm1_ref[...])
    z = _silu(_mm(hg, wf0_ref[0:D, :]) + _mm(hm, wf0_ref[D:2 * D, :]) + bf0_ref[...])
    z = _silu(_mm(z, wf1_ref[...]) + bf1_ref[...])
    out_ref[...] = _mm(z, wf2_ref[...]) + bf2_ref[...]


def _row_spec(w):
    return pl.BlockSpec((BLK, w), lambda i: (i, 0))


def _full_spec(shape):
    nd = len(shape)
    return pl.BlockSpec(shape, lambda i: (0,) * nd)


def _k0(x, wr, b, wa):
    return pl.pallas_call(
        _k0_body,
        grid=(GRID,),
        in_specs=[_row_spec(D), _full_spec((D, D)), _full_spec((1, D)),
                  _full_spec((D, R * D))],
        out_specs=[_row_spec(D), _row_spec(R * D)],
        out_shape=[jax.ShapeDtypeStruct((N, D), jnp.float32),
                   jax.ShapeDtypeStruct((N, R * D), jnp.float32)],
    )(x, wr, b, wa)


def _row_spec2(w):
    return pl.BlockSpec((BLK, w), lambda p, i: (i, 0))


def _full_spec2(shape):
    nd = len(shape)
    return pl.BlockSpec(shape, lambda p, i: (0,) * nd)


def _k1(pr, p0, p1, g, bb, wr, b1, wa):
    return pl.pallas_call(
        _k1_body,
        grid=(2, GRID),
        in_specs=[_row_spec2(D), _row_spec2(D), _row_spec2(D),
                  _full_spec2((1, D)), _full_spec2((1, D)),
                  _full_spec2((D, D)), _full_spec2((1, D)),
                  _full_spec2((D, R * D))],
        out_specs=[_row_spec2(R * D), _row_spec2(D)],
        out_shape=[jax.ShapeDtypeStruct((N, R * D), jnp.float32),
                   jax.ShapeDtypeStruct((N, D), jnp.float32)],
        scratch_shapes=[pltpu.VMEM((N, D), jnp.float32),
                        pltpu.VMEM((2, D), jnp.float32)],
    )(pr, p0, p1, g, bb, wr, b1, wa)


def _k2(pr, p0, p1, wr, b, wa):
    return pl.pallas_call(
        _k2_body,
        grid=(GRID,),
        in_specs=[_row_spec(D), _row_spec(D), _row_spec(D), _full_spec((D, D)),
                  _full_spec((1, D)), _full_spec((D, R * D))],
        out_specs=[_row_spec(R * D), _row_spec(D)],
        out_shape=[jax.ShapeDtypeStruct((N, R * D), jnp.float32),
                   jax.ShapeDtypeStruct((N, D), jnp.float32)],
    )(pr, p0, p1, wr, b, wa)


def _k3(pr, p0, p1):
    return pl.pallas_call(
        _k3_body,
        grid=(GRID,),
        in_specs=[_row_spec(D), _row_spec(D), _row_spec(D)],
        out_specs=_row_spec(D),
        out_shape=jax.ShapeDtypeStruct((N, D), jnp.float32),
    )(pr, p0, p1)


def _head(hg0, hg1, mf, pm, bn_m, pf):
    MF = mf.shape[1]
    args = (hg0, hg1, mf,
            pm[0]["w"], pm[0]["b"].reshape(1, -1),
            bn_m["g"].reshape(1, -1), bn_m["b"].reshape(1, -1),
            pm[1]["w"], pm[1]["b"].reshape(1, -1),
            pf[0]["w"], pf[0]["b"].reshape(1, -1),
            pf[1]["w"], pf[1]["b"].reshape(1, -1),
            pf[2]["w"], pf[2]["b"].reshape(1, -1))
    return pl.pallas_call(
        _head_body,
        out_shape=jax.ShapeDtypeStruct((B, 1), jnp.float32),
    )(*args)


def kernel(x, edge_index, edge_type, batch, mol_feats, params):
    x = x.astype(jnp.float32)
    src = edge_index[0].astype(jnp.int32)
    dst = edge_index[1].astype(jnp.int32)
    et = edge_type.astype(jnp.int32)
    batch = batch.astype(jnp.int32)

    ridx = R * src + et
    key = R * dst + et
    pidx3 = (lax.shift_left(ridx, 14) | dst).reshape(NW, NCHUNK, C)

    gc = params["gc"]
    w_all = [jnp.concatenate([p["w_rel"][r] for r in range(R)], axis=1)
             for p in gc]
    b_ = [p["b"].reshape(1, D) for p in gc]

    inv = _count_kernel(key.reshape(NS, NCHUNK_CNT, C))

    pre_root1, y1 = _k0(x, gc[0]["w_root"], b_[0], w_all[0])
    acc1 = _edge_kernel(y1.reshape(4 * N, D), pidx3, inv)

    y2, pre_root2 = _k1(pre_root1, acc1[0], acc1[1],
                        params["bn_gc"]["g"].reshape(1, D),
                        params["bn_gc"]["b"].reshape(1, D),
                        gc[1]["w_root"], b_[1], w_all[1])
    acc2 = _edge_kernel(y2.reshape(4 * N, D), pidx3, inv)

    y3, pre_root3 = _k2(pre_root2, acc2[0], acc2[1], gc[2]["w_root"], b_[2],
                        w_all[2])
    acc3 = _edge_kernel(y3.reshape(4 * N, D), pidx3, inv)
    h3 = _k3(pre_root3, acc3[0], acc3[1])

    hg = _pool_kernel(h3, batch)
    return _head(hg[0], hg[1], mol_feats.astype(jnp.float32),
                 params["fc_m"], params["bn_m"], params["fc"])


# speedup vs baseline: 1.0190x; 1.0190x over previous
import functools

import jax
import jax.numpy as jnp
from jax import lax
from jax.experimental import pallas as pl
from jax.experimental.pallas import tpu as pltpu
from jax.experimental.pallas import tpu_sc as plsc

N = 10000
E = 320000
D = 128
R = 4
B = 512
NC = 2
NS = 16
L = 16
NW = NC * NS
EPT = E // NW
C = 80
NCHUNK = EPT // C
ROW_STRIDE = 624
ROW_SPAN = 640
CNT_SLICE = (4 * N) // 10
EPC = E // NS
NCHUNK_CNT = EPC // C

_mesh = plsc.VectorSubcoreMesh(core_axis_name="c", subcore_axis_name="s")


@functools.partial(
    pl.kernel,
    mesh=_mesh,
    out_type=jax.ShapeDtypeStruct((4 * N,), jnp.float32),
    scratch_types=[
        pltpu.VMEM((NCHUNK_CNT, C), jnp.int32),
        pltpu.VMEM((C,), jnp.float32),
        pltpu.VMEM((CNT_SLICE,), jnp.float32),
        pltpu.VMEM_SHARED((4 * N,), jnp.float32),
    ],
)
def _count_kernel(key_hbm, inv_hbm, keys_v, ones_v, buf_v, cnt_sh):
    s = lax.axis_index("s")

    def zero_body(i, _):
        buf_v[pl.ds(i * L, L)] = jnp.zeros((L,), jnp.float32)
        return 0

    lax.fori_loop(0, CNT_SLICE // L, zero_body, 0)

    @pl.when(s < 10)
    def _():
        pltpu.sync_copy(buf_v, cnt_sh.at[pl.ds(s * CNT_SLICE, CNT_SLICE)])

    for i in range(C // L):
        ones_v[pl.ds(i * L, L)] = jnp.ones((L,), jnp.float32)
    pltpu.sync_copy(key_hbm.at[s], keys_v)
    plsc.subcore_barrier()

    def chunk(k, _):
        pltpu.sync_copy(ones_v, cnt_sh.at[keys_v.at[k]], add=True)
        return 0

    lax.fori_loop(0, NCHUNK_CNT, chunk, 0)
    plsc.subcore_barrier()

    @pl.when(s < 10)
    def _():
        pltpu.sync_copy(cnt_sh.at[pl.ds(s * CNT_SLICE, CNT_SLICE)], buf_v)

        def inv_body(i, _):
            sl = pl.ds(i * L, L)
            buf_v[sl] = 1.0 / jnp.maximum(buf_v[sl], 1.0)
            return 0

        lax.fori_loop(0, CNT_SLICE // L, inv_body, 0)
        pltpu.sync_copy(buf_v, inv_hbm.at[pl.ds(s * CNT_SLICE, CNT_SLICE)])


@functools.partial(
    pl.kernel,
    mesh=_mesh,
    out_type=jax.ShapeDtypeStruct((NC, N, D), jnp.float32),
    scratch_types=[
        pltpu.VMEM((NCHUNK, C), jnp.int32),
        pltpu.VMEM((C,), jnp.int32), pltpu.VMEM((C,), jnp.int32),
        pltpu.VMEM((C,), jnp.int32),
        pltpu.VMEM((C,), jnp.int32), pltpu.VMEM((C,), jnp.int32),
        pltpu.VMEM((C,), jnp.int32),
        pltpu.VMEM((C,), jnp.int32), pltpu.VMEM((C,), jnp.int32),
        pltpu.VMEM((C,), jnp.int32),
        pltpu.VMEM((C,), jnp.float32), pltpu.VMEM((C,), jnp.float32),
        pltpu.VMEM((C,), jnp.float32),
        pltpu.VMEM((C, D), jnp.float32), pltpu.VMEM((C, D), jnp.float32),
        pltpu.VMEM((C, D), jnp.float32),
        pltpu.VMEM_SHARED((N, D), jnp.float32),
        pltpu.SemaphoreType.DMA, pltpu.SemaphoreType.DMA,
        pltpu.SemaphoreType.DMA,
        pltpu.SemaphoreType.DMA, pltpu.SemaphoreType.DMA,
        pltpu.SemaphoreType.DMA,
        pltpu.SemaphoreType.DMA, pltpu.SemaphoreType.DMA,
        pltpu.SemaphoreType.DMA,
    ],
)
def _edge_kernel(y_hbm, pidx_hbm, inv_hbm, out_hbm,
                 pidx_v, ridx0, ridx1, ridx2, dst0, dst1, dst2,
                 key0, key1, key2, svl0, svl1, svl2,
                 rows0, rows1, rows2, acc_sh,
                 g0, g1, g2, v0, v1, v2, s0, s1, s2):
    c = lax.axis_index("c")
    s = lax.axis_index("s")
    wid = c * NS + s
    RIDX = (ridx0, ridx1, ridx2)
    DST = (dst0, dst1, dst2)
    KEY = (key0, key1, key2)
    SVL = (svl0, svl1, svl2)
    ROWS = (rows0, rows1, rows2)
    G = (g0, g1, g2)
    V = (v0, v1, v2)
    SS = (s0, s1, s2)

    def zero_body(i, _):
        rows0[i >> 3, pl.ds((i & 7) * L, L)] = jnp.zeros((L,), jnp.float32)
        return 0

    lax.fori_loop(0, C * (D // L), zero_body, 0)
    for t in range(ROW_SPAN // C):
        pltpu.sync_copy(rows0,
                        acc_sh.at[pl.ds(s * ROW_STRIDE + t * C, C)])

    pltpu.sync_copy(pidx_hbm.at[wid], pidx_v)
    plsc.subcore_barrier()

    def launch(k, b):
        def group(g, _):
            sl = pl.ds(g * L, L)
            p16 = pidx_v[k, sl]
            d16 = lax.bitwise_and(p16, (1 << 14) - 1)
            r16 = lax.shift_right_logical(p16, 14)
            RIDX[b][sl] = r16
            DST[b][sl] = d16
            KEY[b][sl] = 4 * d16 + lax.bitwise_and(r16, 3)
            return 0

        lax.fori_loop(0, C // L, group, 0)
        pltpu.async_copy(y_hbm.at[RIDX[b]], ROWS[b], G[b])
        pltpu.async_copy(inv_hbm.at[KEY[b]], SVL[b], V[b])

    def wait_g(b):
        pltpu.make_async_copy(y_hbm.at[RIDX[b]], ROWS[b], G[b]).wait()

    def wait_v(b):
        pltpu.make_async_copy(inv_hbm.at[KEY[b]], SVL[b], V[b]).wait()

    def wait_s(b):
        pltpu.make_async_copy(ROWS[b], acc_sh.at[DST[b]], SS[b]).wait()

    def scale(b):
        def group(g, _):
            sv16 = SVL[b][pl.ds(g * L, L)]
            for e in range(L):
                sb = jnp.broadcast_to(sv16[e], (L,))
                row = g * L + e
                for j in range(D // L):
                    sl = pl.ds(j * L, L)
                    ROWS[b][row, sl] = ROWS[b][row, sl] * sb
            return 0

        lax.fori_loop(0, C // L, group, 0)

    def step(j, b, bn, first_steps):
        if not first_steps:
            wait_s(bn)

        @pl.when(j + 1 < NCHUNK)
        def _():
            launch(j + 1, bn)

        wait_g(b)
        wait_v(b)
        scale(b)
        pltpu.async_copy(ROWS[b], acc_sh.at[DST[b]], SS[b], add=True)

    launch(0, 0)

    def triple(m, _):
        j = 3 * m
        for jj in range(3):
            step(j + jj, jj, (jj + 1) % 3, False)
        return 0

    for jj in range(3):
        step(jj, jj, (jj + 1) % 3, jj < 2)
    lax.fori_loop(1, (NCHUNK - 2) // 3, triple, 0)

    wait_s(1)
    launch(NCHUNK - 1, 1)
    wait_g(0)
    wait_v(0)
    scale(0)
    pltpu.async_copy(ROWS[0], acc_sh.at[DST[0]], SS[0], add=True)

    wait_s(2)
    wait_g(1)
    wait_v(1)
    scale(1)
    pltpu.sync_copy(ROWS[1], acc_sh.at[DST[1]], add=True)
    wait_s(0)

    plsc.subcore_barrier()
    pltpu.sync_copy(acc_sh.at[pl.ds(s * ROW_STRIDE, ROW_SPAN)],
                    out_hbm.at[c, pl.ds(s * ROW_STRIDE, ROW_SPAN)])


POOL_CHUNKS = N // C
POOL_ITERS = -(-POOL_CHUNKS // NW)
B_PER_TILE = B // NS


@functools.partial(
    pl.kernel,
    mesh=_mesh,
    out_type=jax.ShapeDtypeStruct((NC, B, D), jnp.float32),
    scratch_types=[
        pltpu.VMEM((C,), jnp.int32),
        pltpu.VMEM((C, D), jnp.float32),
        pltpu.VMEM_SHARED((B, D), jnp.float32),
    ],
)
def _pool_kernel(h_hbm, batch_hbm, out_hbm, bat_v, rows_v, acc_sh):
    c = lax.axis_index("c")
    s = lax.axis_index("s")
    wid = c * NS + s

    def zero_body(i, _):
        rows_v[i >> 3, pl.ds((i & 7) * L, L)] = jnp.zeros((L,), jnp.float32)
        return 0

    lax.fori_loop(0, B_PER_TILE * (D // L), zero_body, 0)
    pltpu.sync_copy(rows_v.at[pl.ds(0, B_PER_TILE)],
                    acc_sh.at[pl.ds(s * B_PER_TILE, B_PER_TILE)])
    plsc.subcore_barrier()

    def body(k, _):
        cid = k * NW + wid

        @pl.when(cid < POOL_CHUNKS)
        def _():
            base = cid * C
            pltpu.sync_copy(batch_hbm.at[pl.ds(base, C)], bat_v)
            pltpu.sync_copy(h_hbm.at[pl.ds(base, C)], rows_v)
            pltpu.sync_copy(rows_v, acc_sh.at[bat_v], add=True)

        return 0

    lax.fori_loop(0, POOL_ITERS, body, 0)
    plsc.subcore_barrier()
    pltpu.sync_copy(acc_sh.at[pl.ds(s * B_PER_TILE, B_PER_TILE)],
                    out_hbm.at[c, pl.ds(s * B_PER_TILE, B_PER_TILE)])


BLK = 1000
GRID = N // BLK


def _silu(v):
    return v / (1.0 + jnp.exp(-v))


def _mm(a, w):
    return jnp.dot(a, w, preferred_element_type=jnp.float32)


def _k0_body(x_ref, wr_ref, b_ref, wa_ref, pre_ref, y_ref):
    xb = x_ref[...]
    pre_ref[...] = _mm(xb, wr_ref[...]) + b_ref[...]
    y_ref[...] = _mm(xb, wa_ref[...])


def _k1_body(pr_ref, p0_ref, p1_ref, g_ref, bb_ref, wr_ref, b1_ref, wa_ref,
             y_ref, pr2_ref, pre_buf, acc_ref):
    p = pl.program_id(0)
    i = pl.program_id(1)

    @pl.when(jnp.logical_and(p == 0, i == 0))
    def _():
        acc_ref[...] = jnp.zeros_like(acc_ref)

    @pl.when(p == 0)
    def _():
        v = pr_ref[...] + p0_ref[...] + p1_ref[...]
        pre_buf[pl.ds(i * BLK, BLK), :] = v
        s1 = jnp.sum(v, axis=0, keepdims=True)
        s2 = jnp.sum(v * v, axis=0, keepdims=True)
        acc_ref[...] = acc_ref[...] + jnp.concatenate([s1, s2], axis=0)

    @pl.when(p == 1)
    def _():
        st = acc_ref[...]
        m = st[0:1] / N
        var = st[1:2] / N - m * m
        v = ((pre_buf[pl.ds(i * BLK, BLK), :] - m) / jnp.sqrt(var + 1e-5)
             * g_ref[...] + bb_ref[...])
        h = _silu(v)
        y_ref[...] = _mm(h, wa_ref[...])
        pr2_ref[...] = _mm(h, wr_ref[...]) + b1_ref[...]


def _k2_body(pr_ref, p0_ref, p1_ref, wr_ref, b_ref, wa_ref, y_ref, pr2_ref):
    h = _silu(pr_ref[...] + p0_ref[...] + p1_ref[...])
    y_ref[...] = _mm(h, wa_ref[...])
    pr2_ref[...] = _mm(h, wr_ref[...]) + b_ref[...]


def _k3_body(pr_ref, p0_ref, p1_ref, h_ref):
    h_ref[...] = _silu(pr_ref[...] + p0_ref[...] + p1_ref[...])


def _head_body(hg0_ref, hg1_ref, mf_ref, wm0_ref, bm0_ref, gm_ref, bbm_ref,
               wm1_ref, bm1_ref, wf0_ref, bf0_ref, wf1_ref, bf1_ref,
               wf2_ref, bf2_ref, out_ref):
    hg = hg0_ref[...] + hg1_ref[...]
    hm = _mm(mf_ref[...], wm0_ref[...]) + bm0_ref[...]
    m = jnp.mean(hm, axis=0, keepdims=True)
    var = jnp.mean((hm - m) ** 2, axis=0, keepdims=True)
    hm = (hm - m) / jnp.sqrt(var + 1e-5) * gm_ref[...] + bbm_ref[...]
    hm = _silu(hm)
    hm = _silu(_mm(hm, wm1_ref[...]) + bm1_ref[...])
    z = _silu(_mm(hg, wf0_ref[0:D, :]) + _mm(hm, wf0_ref[D:2 * D, :]) + bf0_ref[...])
    z = _silu(_mm(z, wf1_ref[...]) + bf1_ref[...])
    out_ref[...] = _mm(z, wf2_ref[...]) + bf2_ref[...]


def _row_spec(w):
    return pl.BlockSpec((BLK, w), lambda i: (i, 0))


def _full_spec(shape):
    nd = len(shape)
    return pl.BlockSpec(shape, lambda i: (0,) * nd)


def _k0(x, wr, b, wa):
    return pl.pallas_call(
        _k0_body,
        grid=(GRID,),
        in_specs=[_row_spec(D), _full_spec((D, D)), _full_spec((1, D)),
                  _full_spec((D, R * D))],
        out_specs=[_row_spec(D), _row_spec(R * D)],
        out_shape=[jax.ShapeDtypeStruct((N, D), jnp.float32),
                   jax.ShapeDtypeStruct((N, R * D), jnp.float32)],
    )(x, wr, b, wa)


def _row_spec2(w):
    return pl.BlockSpec((BLK, w), lambda p, i: ((1 - p) * i, 0))


def _out_spec2(w):
    return pl.BlockSpec((BLK, w), lambda p, i: (p * i, 0))


def _full_spec2(shape):
    nd = len(shape)
    return pl.BlockSpec(shape, lambda p, i: (0,) * nd)


def _k1(pr, p0, p1, g, bb, wr, b1, wa):
    return pl.pallas_call(
        _k1_body,
        grid=(2, GRID),
        in_specs=[_row_spec2(D), _row_spec2(D), _row_spec2(D),
                  _full_spec2((1, D)), _full_spec2((1, D)),
                  _full_spec2((D, D)), _full_spec2((1, D)),
                  _full_spec2((D, R * D))],
        out_specs=[_out_spec2(R * D), _out_spec2(D)],
        out_shape=[jax.ShapeDtypeStruct((N, R * D), jnp.float32),
                   jax.ShapeDtypeStruct((N, D), jnp.float32)],
        scratch_shapes=[pltpu.VMEM((N, D), jnp.float32),
                        pltpu.VMEM((2, D), jnp.float32)],
    )(pr, p0, p1, g, bb, wr, b1, wa)


def _k2(pr, p0, p1, wr, b, wa):
    return pl.pallas_call(
        _k2_body,
        grid=(GRID,),
        in_specs=[_row_spec(D), _row_spec(D), _row_spec(D), _full_spec((D, D)),
                  _full_spec((1, D)), _full_spec((D, R * D))],
        out_specs=[_row_spec(R * D), _row_spec(D)],
        out_shape=[jax.ShapeDtypeStruct((N, R * D), jnp.float32),
                   jax.ShapeDtypeStruct((N, D), jnp.float32)],
    )(pr, p0, p1, wr, b, wa)


def _k3(pr, p0, p1):
    return pl.pallas_call(
        _k3_body,
        grid=(GRID,),
        in_specs=[_row_spec(D), _row_spec(D), _row_spec(D)],
        out_specs=_row_spec(D),
        out_shape=jax.ShapeDtypeStruct((N, D), jnp.float32),
    )(pr, p0, p1)


def _head(hg0, hg1, mf, pm, bn_m, pf):
    MF = mf.shape[1]
    args = (hg0, hg1, mf,
            pm[0]["w"], pm[0]["b"].reshape(1, -1),
            bn_m["g"].reshape(1, -1), bn_m["b"].reshape(1, -1),
            pm[1]["w"], pm[1]["b"].reshape(1, -1),
            pf[0]["w"], pf[0]["b"].reshape(1, -1),
            pf[1]["w"], pf[1]["b"].reshape(1, -1),
            pf[2]["w"], pf[2]["b"].reshape(1, -1))
    return pl.pallas_call(
        _head_body,
        out_shape=jax.ShapeDtypeStruct((B, 1), jnp.float32),
    )(*args)


def kernel(x, edge_index, edge_type, batch, mol_feats, params):
    x = x.astype(jnp.float32)
    src = edge_index[0].astype(jnp.int32)
    dst = edge_index[1].astype(jnp.int32)
    et = edge_type.astype(jnp.int32)
    batch = batch.astype(jnp.int32)

    ridx = R * src + et
    key = R * dst + et
    pidx3 = (lax.shift_left(ridx, 14) | dst).reshape(NW, NCHUNK, C)

    gc = params["gc"]
    w_all = [jnp.concatenate([p["w_rel"][r] for r in range(R)], axis=1)
             for p in gc]
    b_ = [p["b"].reshape(1, D) for p in gc]

    inv = _count_kernel(key.reshape(NS, NCHUNK_CNT, C))

    pre_root1, y1 = _k0(x, gc[0]["w_root"], b_[0], w_all[0])
    acc1 = _edge_kernel(y1.reshape(4 * N, D), pidx3, inv)

    y2, pre_root2 = _k1(pre_root1, acc1[0], acc1[1],
                        params["bn_gc"]["g"].reshape(1, D),
                        params["bn_gc"]["b"].reshape(1, D),
                        gc[1]["w_root"], b_[1], w_all[1])
    acc2 = _edge_kernel(y2.reshape(4 * N, D), pidx3, inv)

    y3, pre_root3 = _k2(pre_root2, acc2[0], acc2[1], gc[2]["w_root"], b_[2],
                        w_all[2])
    acc3 = _edge_kernel(y3.reshape(4 * N, D), pidx3, inv)
    h3 = _k3(pre_root3, acc3[0], acc3[1])

    hg = _pool_kernel(h3, batch)
    return _head(hg[0], hg[1], mol_feats.astype(jnp.float32),
                 params["fc_m"], params["bn_m"], params["fc"])
